# baseline (speedup 1.0000x reference)
import functools

import jax
import jax.numpy as jnp
from jax import lax
from jax.experimental import pallas as pl
from jax.experimental.pallas import tpu as pltpu
from jax.experimental.pallas import tpu_sc as plsc

_N_FIELDS = 26
_VOCAB = 100000
_DIM = 64
_BATCH = 16384

_NC = 2
_NS = 16
_NW = _NC * _NS

_B_TOTAL = _N_FIELDS * _BATCH
_R = _B_TOTAL // _NW
_C = 128
_NCHUNK = _R // _C
_PAIR_ROWS = _N_FIELDS * _VOCAB // 2

_mesh = plsc.VectorSubcoreMesh(core_axis_name="c", subcore_axis_name="s")


@functools.partial(
    pl.kernel,
    mesh=_mesh,
    compiler_params=pltpu.CompilerParams(
        use_tc_tiling_on_sc=True, needs_layout_passes=False
    ),
    out_type=jax.ShapeDtypeStruct((_DIM, _B_TOTAL), jnp.float32),
    scratch_types=[
        pltpu.VMEM((_NCHUNK, _C), jnp.int32),
        pltpu.VMEM((_NCHUNK, _C), jnp.int32),
        pltpu.VMEM((2 * _C, 2 * _DIM), jnp.float32),
        pltpu.VMEM((2 * _DIM, _C), jnp.float32),
        pltpu.SemaphoreType.DMA,
        pltpu.SemaphoreType.DMA,
    ],
)
def _sc_gather_t(x_hbm, t2_hbm, out_hbm, idx_v, colb_v, gbuf, stage, gsem, osem):
    wid = lax.axis_index("s") * _NC + lax.axis_index("c")
    base_row = wid * _R

    pltpu.sync_copy(x_hbm.at[wid], idx_v)

    def _preprocess(c, carry):
        off = ((base_row + c * _C) // _BATCH) * _VOCAB
        for s in range(_C // 16):
            sl = pl.ds(s * 16, 16)
            g = idx_v[c, sl] + off
            idx_v[c, sl] = g >> 1
            colb_v[c, sl] = (g & 1) << 6
        return carry

    lax.fori_loop(0, _NCHUNK, _preprocess, 0)

    def _fire_gather(c, b):
        pltpu.async_copy(t2_hbm.at[idx_v.at[c]], gbuf.at[pl.ds(b * _C, _C)], gsem)

    def _drain_gather(b):
        pltpu.make_async_copy(
            t2_hbm.at[pl.ds(0, _C)], gbuf.at[pl.ds(b * _C, _C)], gsem
        ).wait()

    def _fire_writeback(c, b):
        pltpu.async_copy(
            stage.at[pl.ds(b * _DIM, _DIM)],
            out_hbm.at[:, pl.ds(base_row + c * _C, _C)],
            osem,
        )

    def _drain_writeback(b):
        pltpu.make_async_copy(
            t2_hbm.at[pl.ds(0, _DIM)],
            stage.at[pl.ds(b * _DIM, _DIM)],
            osem,
        ).wait()

    iota = lax.iota(jnp.int32, 16)
    jvs = [iota + lg * 16 for lg in range(8)]

    def _extract(c, b):
        rvs = [jvs[lg] + b * _C for lg in range(8)]
        cbs = [colb_v[c, pl.ds(lg * 16, 16)] for lg in range(8)]

        def _dbody(d, carry):
            for lg in range(8):
                v = plsc.load_gather(gbuf, [rvs[lg], cbs[lg] + d])
                stage[b * _DIM + d, pl.ds(lg * 16, 16)] = v
            return carry

        lax.fori_loop(0, _DIM, _dbody, 0)

    _fire_gather(0, 0)
    _fire_gather(1, 1)

    _drain_gather(0)
    _extract(0, 0)
    _fire_gather(2, 0)
    _fire_writeback(0, 0)

    _drain_gather(1)
    _extract(1, 1)
    _fire_gather(3, 1)
    _fire_writeback(1, 1)

    def _chunk(c, carry):
        b = c % 2
        _drain_gather(b)
        _drain_writeback(b)
        _extract(c, b)
        _fire_gather(c + 2, b)
        _fire_writeback(c, b)
        return carry

    lax.fori_loop(2, _NCHUNK - 2, _chunk, 0)

    for c in (_NCHUNK - 2, _NCHUNK - 1):
        b = c % 2
        _drain_gather(b)
        _drain_writeback(b)
        _extract(c, b)
        _fire_writeback(c, b)

    _drain_writeback(0)
    _drain_writeback(1)


def kernel(X, tables):
    xr = X.reshape(_NW, _NCHUNK, _C)
    t2 = tables.reshape(_PAIR_ROWS, 2 * _DIM)
    out_t = _sc_gather_t(xr, t2)
    return out_t.T


# speedup vs baseline: 1.5048x; 1.5048x over previous
import functools

import jax
import jax.numpy as jnp
from jax import lax
from jax.experimental import pallas as pl
from jax.experimental.pallas import tpu as pltpu
from jax.experimental.pallas import tpu_sc as plsc

_N_FIELDS = 26
_VOCAB = 100000
_DIM = 64
_BATCH = 16384

_NC = 2
_NS = 16
_NW = _NC * _NS

_B_TOTAL = _N_FIELDS * _BATCH
_R = _B_TOTAL // _NW
_C = 128
_NCHUNK = _R // _C
_T_ROWS = _N_FIELDS * _VOCAB

_mesh = plsc.VectorSubcoreMesh(core_axis_name="c", subcore_axis_name="s")


@functools.partial(
    pl.kernel,
    mesh=_mesh,
    compiler_params=pltpu.CompilerParams(
        use_tc_tiling_on_sc=True, needs_layout_passes=False
    ),
    out_type=jax.ShapeDtypeStruct((_DIM, _B_TOTAL), jnp.float32),
    scratch_types=[
        pltpu.VMEM((_NCHUNK, _C), jnp.int32),
        pltpu.VMEM((2 * _C, 2 * _DIM), jnp.float32),
        pltpu.VMEM((2 * _DIM, _C), jnp.float32),
        pltpu.SemaphoreType.DMA,
        pltpu.SemaphoreType.DMA,
    ],
)
def _sc_gather_t(x_hbm, tp_hbm, out_hbm, idx_v, gbuf, stage, gsem, osem):
    wid = lax.axis_index("s") * _NC + lax.axis_index("c")
    base_row = wid * _R

    pltpu.sync_copy(x_hbm.at[wid], idx_v)

    def _preprocess(c, carry):
        off = ((base_row + c * _C) // _BATCH) * _VOCAB
        for s in range(_C // 16):
            sl = pl.ds(s * 16, 16)
            idx_v[c, sl] = idx_v[c, sl] + off
        return carry

    lax.fori_loop(0, _NCHUNK, _preprocess, 0)

    def _fire_gather(c, b):
        pltpu.async_copy(tp_hbm.at[idx_v.at[c]], gbuf.at[pl.ds(b * _C, _C)], gsem)

    def _drain_gather(b):
        pltpu.make_async_copy(
            tp_hbm.at[pl.ds(0, _C)], gbuf.at[pl.ds(b * _C, _C)], gsem
        ).wait()

    def _fire_writeback(c, b):
        pltpu.async_copy(
            stage.at[pl.ds(b * _DIM, _DIM)],
            out_hbm.at[:, pl.ds(base_row + c * _C, _C)],
            osem,
        )

    def _drain_writeback(b):
        pltpu.make_async_copy(
            tp_hbm.at[pl.ds(0, _DIM)],
            stage.at[pl.ds(b * _DIM, _DIM)],
            osem,
        ).wait()

    iota = lax.iota(jnp.int32, 16)
    jvs = [iota + lg * 16 for lg in range(8)]

    def _extract(b):
        rvs = [jvs[lg] + b * _C for lg in range(8)]
        srow = b * _DIM

        @functools.partial(plsc.parallel_loop, 0, _DIM, unroll=4)
        def _dbody(d):
            dv = jnp.full((16,), d, jnp.int32)
            for lg in range(8):
                v = plsc.load_gather(gbuf, [rvs[lg], dv])
                stage[srow + d, pl.ds(lg * 16, 16)] = v

    _fire_gather(0, 0)
    _fire_gather(1, 1)

    _drain_gather(0)
    _extract(0)
    _fire_gather(2, 0)
    _fire_writeback(0, 0)

    _drain_gather(1)
    _extract(1)
    _fire_gather(3, 1)
    _fire_writeback(1, 1)

    def _chunk(c, carry):
        b = c % 2
        _drain_gather(b)
        _drain_writeback(b)
        _extract(b)
        _fire_gather(c + 2, b)
        _fire_writeback(c, b)
        return carry

    lax.fori_loop(2, _NCHUNK - 2, _chunk, 0)

    for c in (_NCHUNK - 2, _NCHUNK - 1):
        b = c % 2
        _drain_gather(b)
        _drain_writeback(b)
        _extract(b)
        _fire_writeback(c, b)

    _drain_writeback(0)
    _drain_writeback(1)


def kernel(X, tables):
    xr = X.reshape(_NW, _NCHUNK, _C)
    tp = jnp.pad(tables, ((0, 0), (0, 0), (0, _DIM))).reshape(_T_ROWS, 2 * _DIM)
    out_t = _sc_gather_t(xr, tp)
    return out_t.T
